# baseline (speedup 1.0000x reference)
import functools

import jax
import jax.numpy as jnp
from jax import lax
from jax.experimental import pallas as pl
from jax.experimental.pallas import tpu as pltpu
from jax.experimental.pallas import tpu_sc as plsc

_ATANH_HALF = 0.5493061443340549

_NUM_WORKERS = 32
_LANES = 16
_PER_W = 272
_N_CHUNKS = 8


def _sc_mask_decisions(plan_rows, plan_cols, m):
    n_edges = plan_rows.shape[0]
    half_w = _NUM_WORKERS // 2
    per_w = -(-n_edges // half_w)
    ngroups = -(-per_w // _LANES)
    e_pad = half_w * ngroups * _LANES
    e_out = _NUM_WORKERS * _PER_W

    mesh = plsc.VectorSubcoreMesh(core_axis_name="c", subcore_axis_name="s")

    @functools.partial(
        pl.kernel,
        out_type=(
            jax.ShapeDtypeStruct((e_out,), jnp.int32),
            jax.ShapeDtypeStruct((e_out,), jnp.int32),
            jax.ShapeDtypeStruct((e_out,), jnp.float32),
            jax.ShapeDtypeStruct((_NUM_WORKERS, _LANES), jnp.int32),
        ),
        mesh=mesh,
        compiler_params=pltpu.CompilerParams(needs_layout_passes=False),
        scratch_types=[
            pltpu.VMEM((e_pad,), jnp.int32),
            pltpu.VMEM((e_pad,), jnp.int32),
            pltpu.VMEM((e_pad,), jnp.float32),
            pltpu.VMEM((_PER_W + _LANES,), jnp.int32),
            pltpu.VMEM((_PER_W + _LANES,), jnp.int32),
            pltpu.VMEM((_PER_W + _LANES,), jnp.float32),
            pltpu.VMEM((_LANES,), jnp.int32),
            pltpu.SemaphoreType.DMA,
        ],
    )
    def sc_kernel(rows_hbm, cols_hbm, m_hbm,
                  mrow_hbm, mcol_hbm, mval_hbm, cnt_hbm,
                  rv, cv, mv, orow, ocol, oval, cnt_v, lsem):
        cid = lax.axis_index("c")
        sid = lax.axis_index("s")
        wid = sid * 2 + cid
        swapped = wid >= half_w
        base = (wid % half_w) * per_w
        h1 = pltpu.async_copy(rows_hbm, rv.at[pl.ds(0, n_edges)], lsem)
        h2 = pltpu.async_copy(cols_hbm, cv.at[pl.ds(0, n_edges)], lsem)
        h3 = pltpu.async_copy(m_hbm, mv.at[pl.ds(0, n_edges)], lsem)
        h1.wait()
        h2.wait()
        h3.wait()

        lane = lax.iota(jnp.int32, _LANES)

        @pl.loop(0, ngroups, init_carry=jnp.int32(0))
        def scan(g, off):
            b = base + g * _LANES
            ra = rv[pl.ds(b, _LANES)]
            ca = cv[pl.ds(b, _LANES)]
            mm = mv[pl.ds(b, _LANES)]
            r = jnp.where(swapped, ca, ra)
            c = jnp.where(swapped, ra, ca)
            pos = mm > _ATANH_HALF
            valid = (g * _LANES + lane) < per_w
            match = (pos | (mm < -_ATANH_HALF)) & valid
            val = jnp.where(pos, jnp.float32(1.0), jnp.float32(0.0))
            plsc.store_compressed(orow.at[pl.ds(off, _LANES)], r, mask=match)
            plsc.store_compressed(ocol.at[pl.ds(off, _LANES)], c, mask=match)
            plsc.store_compressed(oval.at[pl.ds(off, _LANES)], val, mask=match)
            return off + plsc.all_reduce_population_count(match)[0]

        cnt_v[...] = jnp.full((_LANES,), scan, dtype=jnp.int32)
        obase = wid * _PER_W
        pltpu.sync_copy(orow.at[pl.ds(0, _PER_W)], mrow_hbm.at[pl.ds(obase, _PER_W)])
        pltpu.sync_copy(ocol.at[pl.ds(0, _PER_W)], mcol_hbm.at[pl.ds(obase, _PER_W)])
        pltpu.sync_copy(oval.at[pl.ds(0, _PER_W)], mval_hbm.at[pl.ds(obase, _PER_W)])
        pltpu.sync_copy(cnt_v, cnt_hbm.at[wid])

    return sc_kernel(plan_rows, plan_cols, m)


def _tc_gcn(adj, mrow, mcol, mval, counts, x, W1, b1, W2, b2):
    n = adj.shape[0]
    nclass = W2.shape[1]
    rows_per_chunk = n // _N_CHUNKS

    def body(adj_hbm, mrow_ref, mcol_ref, mval_ref, cnt_ref,
             x_hbm, w1_ref, b1_ref, w2_ref, b2_ref, out_ref,
             adj_v, x_v, deg_s, sems, xsem):
        hx = pltpu.async_copy(x_hbm, x_v, xsem)
        handles = [
            pltpu.async_copy(
                adj_hbm.at[pl.ds(k * rows_per_chunk, rows_per_chunk), :],
                adj_v.at[pl.ds(k * rows_per_chunk, rows_per_chunk), :],
                sems.at[k],
            )
            for k in range(_N_CHUNKS)
        ]
        hx.wait()
        u = jnp.dot(x_v[...], w1_ref[...], preferred_element_type=jnp.float32)
        sub_rows = n // 128
        for k in range(_N_CHUNKS):
            handles[k].wait()
            rs = jnp.sum(adj_v[pl.ds(k * rows_per_chunk, rows_per_chunk), :],
                         axis=1, dtype=jnp.float32)
            deg_s[pl.ds(k * (rows_per_chunk // 128), rows_per_chunk // 128), :] = (
                rs.reshape(rows_per_chunk // 128, 128))

        @pl.loop(0, _NUM_WORKERS)
        def seg(k):
            c = cnt_ref[k, 0]

            @pl.loop(0, c)
            def ent(i):
                j = k * _PER_W + i
                r = mrow_ref[j]
                cc = mcol_ref[j]
                v = mval_ref[j]
                r8 = pl.multiple_of((r // 8) * 8, 8)
                blk = adj_v[pl.ds(r8, 8), :]
                sub = lax.broadcasted_iota(jnp.int32, (8, n), 0)
                lanes = lax.broadcasted_iota(jnp.int32, (8, n), 1)
                hit = (sub == r - r8) & (lanes == cc)
                old = jnp.sum(jnp.where(hit, blk, 0.0))
                adj_v[pl.ds(r8, 8), :] = jnp.where(hit, v, blk)
                dblk = deg_s[...]
                dsub = lax.broadcasted_iota(jnp.int32, (sub_rows, 128), 0)
                dlane = lax.broadcasted_iota(jnp.int32, (sub_rows, 128), 1)
                dhit = (dsub == r // 128) & (dlane == r % 128)
                deg_s[...] = jnp.where(dhit, dblk + (v - old), dblk)

        a = adj_v[...]
        deg = deg_s[...].reshape(n) + 1.0
        s = lax.rsqrt(deg)[:, None]
        v1 = u * s
        p1 = jnp.dot(a, v1, preferred_element_type=jnp.float32) + v1
        h = jnp.maximum(p1 * s + b1_ref[...][None, :], 0.0)
        v2 = jnp.dot(h, w2_ref[...], preferred_element_type=jnp.float32) * s
        p2 = jnp.dot(a, v2, preferred_element_type=jnp.float32) + v2
        o = p2 * s + b2_ref[...][None, :]
        mx = jnp.max(o, axis=1, keepdims=True)
        lse = jnp.log(jnp.sum(jnp.exp(o - mx), axis=1, keepdims=True)) + mx
        out_ref[...] = o - lse

    vspec = pl.BlockSpec(memory_space=pltpu.VMEM)
    sspec = pl.BlockSpec(memory_space=pltpu.SMEM)
    aspec = pl.BlockSpec(memory_space=pl.ANY)
    return pl.pallas_call(
        body,
        out_shape=jax.ShapeDtypeStruct((n, nclass), jnp.float32),
        in_specs=[aspec, sspec, sspec, sspec, sspec,
                  aspec, vspec, vspec, vspec, vspec],
        out_specs=vspec,
        scratch_shapes=[
            pltpu.VMEM((n, n), jnp.float32),
            pltpu.VMEM(x.shape, jnp.float32),
            pltpu.VMEM((n // 128, 128), jnp.float32),
            pltpu.SemaphoreType.DMA((_N_CHUNKS,)),
            pltpu.SemaphoreType.DMA,
        ],
        compiler_params=pltpu.CompilerParams(
            vmem_limit_bytes=60000 * 1024,
        ),
    )(adj, mrow, mcol, mval, counts, x, W1, b1, W2, b2)


def kernel(x, sub_adj, M, plan_rows, plan_cols, W1, b1, W2, b2):
    rows = plan_rows.astype(jnp.int32)
    cols = plan_cols.astype(jnp.int32)
    m = M.astype(jnp.float32)
    mrow, mcol, mval, counts = _sc_mask_decisions(rows, cols, m)
    return _tc_gcn(sub_adj, mrow, mcol, mval, counts, x, W1, b1, W2, b2)


# speedup vs baseline: 1.0258x; 1.0258x over previous
import functools

import jax
import jax.numpy as jnp
from jax import lax
from jax.experimental import pallas as pl
from jax.experimental.pallas import tpu as pltpu
from jax.experimental.pallas import tpu_sc as plsc

_ATANH_HALF = 0.5493061443340549

_NUM_WORKERS = 32
_LANES = 16
_PER_W = 272
_N_CHUNKS = 8


def _sc_mask_decisions(plan_rows, plan_cols, m):
    n_edges = plan_rows.shape[0]
    half_w = _NUM_WORKERS // 2
    per_w = -(-n_edges // half_w)
    ngroups = -(-per_w // _LANES)
    e_pad = half_w * ngroups * _LANES
    e_out = _NUM_WORKERS * _PER_W

    mesh = plsc.VectorSubcoreMesh(core_axis_name="c", subcore_axis_name="s")

    @functools.partial(
        pl.kernel,
        out_type=(
            jax.ShapeDtypeStruct((e_out,), jnp.int32),
            jax.ShapeDtypeStruct((e_out,), jnp.int32),
            jax.ShapeDtypeStruct((e_out,), jnp.float32),
            jax.ShapeDtypeStruct((_NUM_WORKERS, _LANES), jnp.int32),
        ),
        mesh=mesh,
        compiler_params=pltpu.CompilerParams(needs_layout_passes=False),
        scratch_types=[
            pltpu.VMEM((e_pad,), jnp.int32),
            pltpu.VMEM((e_pad,), jnp.int32),
            pltpu.VMEM((e_pad,), jnp.float32),
            pltpu.VMEM((_PER_W + _LANES,), jnp.int32),
            pltpu.VMEM((_PER_W + _LANES,), jnp.int32),
            pltpu.VMEM((_PER_W + _LANES,), jnp.float32),
            pltpu.VMEM((_LANES,), jnp.int32),
            pltpu.SemaphoreType.DMA,
        ],
    )
    def sc_kernel(rows_hbm, cols_hbm, m_hbm,
                  mrow_hbm, mcol_hbm, mval_hbm, cnt_hbm,
                  rv, cv, mv, orow, ocol, oval, cnt_v, lsem):
        cid = lax.axis_index("c")
        sid = lax.axis_index("s")
        wid = sid * 2 + cid
        swapped = wid >= half_w
        base = (wid % half_w) * per_w
        h1 = pltpu.async_copy(rows_hbm, rv.at[pl.ds(0, n_edges)], lsem)
        h2 = pltpu.async_copy(cols_hbm, cv.at[pl.ds(0, n_edges)], lsem)
        h3 = pltpu.async_copy(m_hbm, mv.at[pl.ds(0, n_edges)], lsem)
        h1.wait()
        h2.wait()
        h3.wait()

        lane = lax.iota(jnp.int32, _LANES)

        @pl.loop(0, ngroups, init_carry=jnp.int32(0))
        def scan(g, off):
            b = base + g * _LANES
            ra = rv[pl.ds(b, _LANES)]
            ca = cv[pl.ds(b, _LANES)]
            mm = mv[pl.ds(b, _LANES)]
            r = jnp.where(swapped, ca, ra)
            c = jnp.where(swapped, ra, ca)
            pos = mm > _ATANH_HALF
            valid = (g * _LANES + lane) < per_w
            match = (pos | (mm < -_ATANH_HALF)) & valid
            val = jnp.where(pos, jnp.float32(1.0), jnp.float32(0.0))
            plsc.store_compressed(orow.at[pl.ds(off, _LANES)], r, mask=match)
            plsc.store_compressed(ocol.at[pl.ds(off, _LANES)], c, mask=match)
            plsc.store_compressed(oval.at[pl.ds(off, _LANES)], val, mask=match)
            return off + plsc.all_reduce_population_count(match)[0]

        cnt_v[...] = jnp.full((_LANES,), scan, dtype=jnp.int32)
        obase = wid * _PER_W
        pltpu.sync_copy(orow.at[pl.ds(0, _PER_W)], mrow_hbm.at[pl.ds(obase, _PER_W)])
        pltpu.sync_copy(ocol.at[pl.ds(0, _PER_W)], mcol_hbm.at[pl.ds(obase, _PER_W)])
        pltpu.sync_copy(oval.at[pl.ds(0, _PER_W)], mval_hbm.at[pl.ds(obase, _PER_W)])
        pltpu.sync_copy(cnt_v, cnt_hbm.at[wid])

    return sc_kernel(plan_rows, plan_cols, m)


def _tc_gcn(adj, mrow, mcol, mval, counts, x, W1, b1, W2, b2):
    n = adj.shape[0]
    nclass = W2.shape[1]
    rows_per_chunk = n // _N_CHUNKS

    def body(adj_hbm, mrow_ref, mcol_ref, mval_ref, cnt_ref,
             x_hbm, w1_ref, b1_ref, w2_ref, b2_ref, out_ref,
             adj_v, x_v, sems, xsem):
        hx = pltpu.async_copy(x_hbm, x_v, xsem)
        handles = [
            pltpu.async_copy(
                adj_hbm.at[pl.ds(k * rows_per_chunk, rows_per_chunk), :],
                adj_v.at[pl.ds(k * rows_per_chunk, rows_per_chunk), :],
                sems.at[k],
            )
            for k in range(_N_CHUNKS)
        ]
        hx.wait()
        u = jnp.dot(x_v[...], w1_ref[...], preferred_element_type=jnp.float32)
        for k in range(_N_CHUNKS):
            handles[k].wait()

        @pl.loop(0, _NUM_WORKERS)
        def seg(k):
            c = cnt_ref[k, 0]

            @pl.loop(0, c)
            def ent(i):
                j = k * _PER_W + i
                r = mrow_ref[j]
                cc = mcol_ref[j]
                v = mval_ref[j]
                r8 = pl.multiple_of((r // 8) * 8, 8)
                blk = adj_v[pl.ds(r8, 8), :]
                sub = lax.broadcasted_iota(jnp.int32, (8, n), 0)
                lanes = lax.broadcasted_iota(jnp.int32, (8, n), 1)
                hit = (sub == r - r8) & (lanes == cc)
                adj_v[pl.ds(r8, 8), :] = jnp.where(hit, v, blk)

        a = adj_v[...]
        deg = jnp.sum(a, axis=1, dtype=jnp.float32) + 1.0
        s = lax.rsqrt(deg)[:, None]
        v1 = u * s
        p1 = jnp.dot(a, v1, preferred_element_type=jnp.float32) + v1
        h = jnp.maximum(p1 * s + b1_ref[...][None, :], 0.0)
        v2 = jnp.dot(h, w2_ref[...], preferred_element_type=jnp.float32) * s
        p2 = jnp.dot(a, v2, preferred_element_type=jnp.float32) + v2
        o = p2 * s + b2_ref[...][None, :]
        mx = jnp.max(o, axis=1, keepdims=True)
        lse = jnp.log(jnp.sum(jnp.exp(o - mx), axis=1, keepdims=True)) + mx
        out_ref[...] = o - lse

    vspec = pl.BlockSpec(memory_space=pltpu.VMEM)
    sspec = pl.BlockSpec(memory_space=pltpu.SMEM)
    aspec = pl.BlockSpec(memory_space=pl.ANY)
    return pl.pallas_call(
        body,
        out_shape=jax.ShapeDtypeStruct((n, nclass), jnp.float32),
        in_specs=[aspec, sspec, sspec, sspec, sspec,
                  aspec, vspec, vspec, vspec, vspec],
        out_specs=vspec,
        scratch_shapes=[
            pltpu.VMEM((n, n), jnp.float32),
            pltpu.VMEM(x.shape, jnp.float32),
            pltpu.SemaphoreType.DMA((_N_CHUNKS,)),
            pltpu.SemaphoreType.DMA,
        ],
        compiler_params=pltpu.CompilerParams(
            vmem_limit_bytes=60000 * 1024,
        ),
    )(adj, mrow, mcol, mval, counts, x, W1, b1, W2, b2)


def kernel(x, sub_adj, M, plan_rows, plan_cols, W1, b1, W2, b2):
    rows = plan_rows.astype(jnp.int32)
    cols = plan_cols.astype(jnp.int32)
    m = M.astype(jnp.float32)
    mrow, mcol, mval, counts = _sc_mask_decisions(rows, cols, m)
    return _tc_gcn(sub_adj, mrow, mcol, mval, counts, x, W1, b1, W2, b2)


# speedup vs baseline: 1.0729x; 1.0460x over previous
import functools

import jax
import jax.numpy as jnp
from jax import lax
from jax.experimental import pallas as pl
from jax.experimental.pallas import tpu as pltpu
from jax.experimental.pallas import tpu_sc as plsc

_ATANH_HALF = 0.5493061443340549

_NUM_WORKERS = 32
_LANES = 16
_PER_W = 272
_N_CHUNKS = 8


def _sc_mask_decisions(plan_rows, plan_cols, m):
    n_edges = plan_rows.shape[0]
    half_w = _NUM_WORKERS // 2
    per_w = -(-n_edges // half_w)
    ngroups = -(-per_w // _LANES)
    e_pad = half_w * ngroups * _LANES
    e_out = _NUM_WORKERS * _PER_W

    mesh = plsc.VectorSubcoreMesh(core_axis_name="c", subcore_axis_name="s")

    @functools.partial(
        pl.kernel,
        out_type=(
            jax.ShapeDtypeStruct((e_out,), jnp.int32),
            jax.ShapeDtypeStruct((e_out,), jnp.int32),
            jax.ShapeDtypeStruct((e_out,), jnp.float32),
            jax.ShapeDtypeStruct((_NUM_WORKERS, _LANES), jnp.int32),
        ),
        mesh=mesh,
        compiler_params=pltpu.CompilerParams(needs_layout_passes=False),
        scratch_types=[
            pltpu.VMEM((e_pad,), jnp.int32),
            pltpu.VMEM((e_pad,), jnp.int32),
            pltpu.VMEM((e_pad,), jnp.float32),
            pltpu.VMEM((_PER_W + _LANES,), jnp.int32),
            pltpu.VMEM((_PER_W + _LANES,), jnp.int32),
            pltpu.VMEM((_PER_W + _LANES,), jnp.float32),
            pltpu.VMEM((_LANES,), jnp.int32),
            pltpu.SemaphoreType.DMA,
        ],
    )
    def sc_kernel(rows_hbm, cols_hbm, m_hbm,
                  mrow_hbm, mcol_hbm, mval_hbm, cnt_hbm,
                  rv, cv, mv, orow, ocol, oval, cnt_v, lsem):
        cid = lax.axis_index("c")
        sid = lax.axis_index("s")
        wid = sid * 2 + cid
        swapped = wid >= half_w
        base = (wid % half_w) * per_w
        h1 = pltpu.async_copy(rows_hbm, rv.at[pl.ds(0, n_edges)], lsem)
        h2 = pltpu.async_copy(cols_hbm, cv.at[pl.ds(0, n_edges)], lsem)
        h3 = pltpu.async_copy(m_hbm, mv.at[pl.ds(0, n_edges)], lsem)
        h1.wait()
        h2.wait()
        h3.wait()

        lane = lax.iota(jnp.int32, _LANES)

        @pl.loop(0, ngroups, init_carry=jnp.int32(0))
        def scan(g, off):
            b = base + g * _LANES
            ra = rv[pl.ds(b, _LANES)]
            ca = cv[pl.ds(b, _LANES)]
            mm = mv[pl.ds(b, _LANES)]
            r = jnp.where(swapped, ca, ra)
            c = jnp.where(swapped, ra, ca)
            pos = mm > _ATANH_HALF
            valid = (g * _LANES + lane) < per_w
            match = (pos | (mm < -_ATANH_HALF)) & valid
            val = jnp.where(pos, jnp.float32(1.0), jnp.float32(0.0))
            plsc.store_compressed(orow.at[pl.ds(off, _LANES)], r, mask=match)
            plsc.store_compressed(ocol.at[pl.ds(off, _LANES)], c, mask=match)
            plsc.store_compressed(oval.at[pl.ds(off, _LANES)], val, mask=match)
            return off + plsc.all_reduce_population_count(match)[0]

        cnt_v[...] = jnp.full((_LANES,), scan, dtype=jnp.int32)
        obase = wid * _PER_W
        pltpu.sync_copy(orow.at[pl.ds(0, _PER_W)], mrow_hbm.at[pl.ds(obase, _PER_W)])
        pltpu.sync_copy(ocol.at[pl.ds(0, _PER_W)], mcol_hbm.at[pl.ds(obase, _PER_W)])
        pltpu.sync_copy(oval.at[pl.ds(0, _PER_W)], mval_hbm.at[pl.ds(obase, _PER_W)])
        pltpu.sync_copy(cnt_v, cnt_hbm.at[wid])

    return sc_kernel(plan_rows, plan_cols, m)


def _tc_gcn(adj, mrow, mcol, mval, counts, x, W1, b1, W2, b2):
    n = adj.shape[0]
    nclass = W2.shape[1]
    rows_per_chunk = n // _N_CHUNKS

    def body(adj_hbm, mrow_ref, mcol_ref, mval_ref, cnt_ref,
             x_ref, w1_ref, b1_ref, w2_ref, b2_ref, out_ref,
             adj_v, sems):
        ha = pltpu.async_copy(adj_hbm, adj_v, sems)
        u = jnp.dot(x_ref[...], w1_ref[...], preferred_element_type=jnp.float32)
        ha.wait()

        @pl.loop(0, _NUM_WORKERS)
        def seg(k):
            c = cnt_ref[k, 0]

            @pl.loop(0, c)
            def ent(i):
                j = k * _PER_W + i
                r = mrow_ref[j]
                cc = mcol_ref[j]
                v = mval_ref[j]
                r8 = pl.multiple_of((r // 8) * 8, 8)
                blk = adj_v[pl.ds(r8, 8), :]
                sub = lax.broadcasted_iota(jnp.int32, (8, n), 0)
                lanes = lax.broadcasted_iota(jnp.int32, (8, n), 1)
                hit = (sub == r - r8) & (lanes == cc)
                adj_v[pl.ds(r8, 8), :] = jnp.where(hit, v, blk)

        a = adj_v[...]
        deg = jnp.sum(a, axis=1, dtype=jnp.float32) + 1.0
        s = lax.rsqrt(deg)[:, None]
        v1 = u * s
        p1 = jnp.dot(a, v1, preferred_element_type=jnp.float32) + v1
        h = jnp.maximum(p1 * s + b1_ref[...][None, :], 0.0)
        v2 = jnp.dot(h, w2_ref[...], preferred_element_type=jnp.float32) * s
        p2 = jnp.dot(a, v2, preferred_element_type=jnp.float32) + v2
        o = p2 * s + b2_ref[...][None, :]
        mx = jnp.max(o, axis=1, keepdims=True)
        lse = jnp.log(jnp.sum(jnp.exp(o - mx), axis=1, keepdims=True)) + mx
        out_ref[...] = o - lse

    vspec = pl.BlockSpec(memory_space=pltpu.VMEM)
    sspec = pl.BlockSpec(memory_space=pltpu.SMEM)
    aspec = pl.BlockSpec(memory_space=pl.ANY)
    return pl.pallas_call(
        body,
        out_shape=jax.ShapeDtypeStruct((n, nclass), jnp.float32),
        in_specs=[aspec, sspec, sspec, sspec, sspec,
                  vspec, vspec, vspec, vspec, vspec],
        out_specs=vspec,
        scratch_shapes=[
            pltpu.VMEM((n, n), jnp.float32),
            pltpu.SemaphoreType.DMA,
        ],
        compiler_params=pltpu.CompilerParams(
            vmem_limit_bytes=60000 * 1024,
        ),
    )(adj, mrow, mcol, mval, counts, x, W1, b1, W2, b2)


def kernel(x, sub_adj, M, plan_rows, plan_cols, W1, b1, W2, b2):
    rows = plan_rows.astype(jnp.int32)
    cols = plan_cols.astype(jnp.int32)
    m = M.astype(jnp.float32)
    mrow, mcol, mval, counts = _sc_mask_decisions(rows, cols, m)
    return _tc_gcn(sub_adj, mrow, mcol, mval, counts, x, W1, b1, W2, b2)


# speedup vs baseline: 1.0737x; 1.0008x over previous
import functools

import jax
import jax.numpy as jnp
from jax import lax
from jax.experimental import pallas as pl
from jax.experimental.pallas import tpu as pltpu
from jax.experimental.pallas import tpu_sc as plsc

_ATANH_HALF = 0.5493061443340549

_NUM_WORKERS = 32
_LANES = 16
_PER_W = 272


def _sc_mask_decisions(plan_rows, plan_cols, m):
    n_edges = plan_rows.shape[0]
    half_w = _NUM_WORKERS // 2
    per_w = -(-n_edges // half_w)
    ngroups = -(-per_w // _LANES)
    e_pad = half_w * ngroups * _LANES
    e_out = _NUM_WORKERS * _PER_W

    mesh = plsc.VectorSubcoreMesh(core_axis_name="c", subcore_axis_name="s")

    @functools.partial(
        pl.kernel,
        out_type=(
            jax.ShapeDtypeStruct((e_out,), jnp.int32),
            jax.ShapeDtypeStruct((e_out,), jnp.int32),
            jax.ShapeDtypeStruct((e_out,), jnp.float32),
            jax.ShapeDtypeStruct((_NUM_WORKERS, _LANES), jnp.int32),
        ),
        mesh=mesh,
        compiler_params=pltpu.CompilerParams(needs_layout_passes=False),
        scratch_types=[
            pltpu.VMEM((e_pad,), jnp.int32),
            pltpu.VMEM((e_pad,), jnp.int32),
            pltpu.VMEM((e_pad,), jnp.float32),
            pltpu.VMEM((_PER_W + _LANES,), jnp.int32),
            pltpu.VMEM((_PER_W + _LANES,), jnp.int32),
            pltpu.VMEM((_PER_W + _LANES,), jnp.float32),
            pltpu.VMEM((_LANES,), jnp.int32),
            pltpu.SemaphoreType.DMA,
        ],
    )
    def sc_kernel(rows_hbm, cols_hbm, m_hbm,
                  mrow_hbm, mcol_hbm, mval_hbm, cnt_hbm,
                  rv, cv, mv, orow, ocol, oval, cnt_v, lsem):
        cid = lax.axis_index("c")
        sid = lax.axis_index("s")
        wid = sid * 2 + cid
        swapped = wid >= half_w
        base = (wid % half_w) * per_w
        h1 = pltpu.async_copy(rows_hbm, rv.at[pl.ds(0, n_edges)], lsem)
        h2 = pltpu.async_copy(cols_hbm, cv.at[pl.ds(0, n_edges)], lsem)
        h3 = pltpu.async_copy(m_hbm, mv.at[pl.ds(0, n_edges)], lsem)
        h1.wait()
        h2.wait()
        h3.wait()

        lane = lax.iota(jnp.int32, _LANES)

        @pl.loop(0, ngroups, init_carry=jnp.int32(0))
        def scan(g, off):
            b = base + g * _LANES
            ra = rv[pl.ds(b, _LANES)]
            ca = cv[pl.ds(b, _LANES)]
            mm = mv[pl.ds(b, _LANES)]
            r = jnp.where(swapped, ca, ra)
            c = jnp.where(swapped, ra, ca)
            pos = mm > _ATANH_HALF
            valid = (g * _LANES + lane) < per_w
            match = (pos | (mm < -_ATANH_HALF)) & valid
            val = jnp.where(pos, jnp.float32(1.0), jnp.float32(0.0))
            plsc.store_compressed(orow.at[pl.ds(off, _LANES)], r, mask=match)
            plsc.store_compressed(ocol.at[pl.ds(off, _LANES)], c, mask=match)
            plsc.store_compressed(oval.at[pl.ds(off, _LANES)], val, mask=match)
            return off + plsc.all_reduce_population_count(match)[0]

        cnt_v[...] = jnp.full((_LANES,), scan, dtype=jnp.int32)
        obase = wid * _PER_W
        pltpu.sync_copy(orow.at[pl.ds(0, _PER_W)], mrow_hbm.at[pl.ds(obase, _PER_W)])
        pltpu.sync_copy(ocol.at[pl.ds(0, _PER_W)], mcol_hbm.at[pl.ds(obase, _PER_W)])
        pltpu.sync_copy(oval.at[pl.ds(0, _PER_W)], mval_hbm.at[pl.ds(obase, _PER_W)])
        pltpu.sync_copy(cnt_v, cnt_hbm.at[wid])

    return sc_kernel(plan_rows, plan_cols, m)


def _tc_gcn(adj, mrow, mcol, mval, counts, x, W1, b1, W2, b2):
    n = adj.shape[0]
    nclass = W2.shape[1]

    def body(adj_hbm, mrow_ref, mcol_ref, mval_ref, cnt_ref,
             x_ref, w1_ref, b1_ref, w2_ref, b2_ref, out_ref,
             adj_v, sems):
        ha = pltpu.async_copy(adj_hbm, adj_v, sems)
        u = jnp.dot(x_ref[...], w1_ref[...], preferred_element_type=jnp.float32)
        ha.wait()

        @pl.loop(0, _NUM_WORKERS)
        def seg(k):
            c = cnt_ref[k, 0]

            @pl.loop(0, c)
            def ent(i):
                j = k * _PER_W + i
                r = mrow_ref[j]
                cc = mcol_ref[j]
                v = mval_ref[j]
                r8 = pl.multiple_of((r // 8) * 8, 8)
                blk = adj_v[pl.ds(r8, 8), :]
                sub = lax.broadcasted_iota(jnp.int32, (8, n), 0)
                lanes = lax.broadcasted_iota(jnp.int32, (8, n), 1)
                hit = (sub == r - r8) & (lanes == cc)
                adj_v[pl.ds(r8, 8), :] = jnp.where(hit, v, blk)

        a = adj_v[...]
        deg = jnp.sum(a, axis=1, dtype=jnp.float32) + 1.0
        s = lax.rsqrt(deg)[:, None]
        v1 = u * s
        p1 = jnp.dot(a, v1, preferred_element_type=jnp.float32) + v1
        h = jnp.maximum(p1 * s + b1_ref[...][None, :], 0.0)
        v2 = jnp.dot(h, w2_ref[...], preferred_element_type=jnp.float32) * s
        p2 = jnp.dot(a, v2, preferred_element_type=jnp.float32) + v2
        o = p2 * s + b2_ref[...][None, :]
        mx = jnp.max(o, axis=1, keepdims=True)
        lse = jnp.log(jnp.sum(jnp.exp(o - mx), axis=1, keepdims=True)) + mx
        out_ref[...] = o - lse

    vspec = pl.BlockSpec(memory_space=pltpu.VMEM)
    sspec = pl.BlockSpec(memory_space=pltpu.SMEM)
    aspec = pl.BlockSpec(memory_space=pl.ANY)
    return pl.pallas_call(
        body,
        out_shape=jax.ShapeDtypeStruct((n, nclass), jnp.float32),
        in_specs=[aspec, sspec, sspec, sspec, sspec,
                  vspec, vspec, vspec, vspec, vspec],
        out_specs=vspec,
        scratch_shapes=[
            pltpu.VMEM((n, n), jnp.float32),
            pltpu.SemaphoreType.DMA,
        ],
        compiler_params=pltpu.CompilerParams(
            vmem_limit_bytes=60000 * 1024,
        ),
    )(adj, mrow, mcol, mval, counts, x, W1, b1, W2, b2)


def kernel(x, sub_adj, M, plan_rows, plan_cols, W1, b1, W2, b2):
    rows = plan_rows.astype(jnp.int32)
    cols = plan_cols.astype(jnp.int32)
    m = M.astype(jnp.float32)
    mrow, mcol, mval, counts = _sc_mask_decisions(rows, cols, m)
    return _tc_gcn(sub_adj, mrow, mcol, mval, counts, x, W1, b1, W2, b2)
